# Initial kernel scaffold; baseline (speedup 1.0000x reference)
import jax
import jax.numpy as jnp
from jax.experimental import pallas as pl


def kernel(x, review_feat, weight_stack, prob_w_stack, review_w_stack, fc_user_w, fc_user_b, fc_item_w, fc_item_b, edge_index, edge_rating):
    raise NotImplementedError("write your pallas kernel here")


# speedup vs baseline: 1.8629x; 1.8629x over previous
import functools

import jax
import jax.numpy as jnp
from jax import lax
from jax.experimental import pallas as pl

N_U = 5000
N_I = 5000
N = N_U + N_I
E = 320000
D = 128
R = 5

BLK = 1000
NBLK = N // BLK


def _final_body(t1_ref, t2_ref, ci_ref, rw_ref, fcw_ref, fcb_ref, out_ref):
    acc = t1_ref[0]
    for r in range(R):
        acc = acc + lax.dot_general(
            t2_ref[r], rw_ref[r],
            (((1,), (1,)), ((), ())),
            preferred_element_type=jnp.float32)
    acc = acc * ci_ref[...]
    out_ref[...] = lax.dot_general(
        acc, fcw_ref[0], (((1,), (1,)), ((), ())),
        preferred_element_type=jnp.float32) + fcb_ref[0]


def _final_stage(t1, t2, ci, review_w_stack, fc_w, fc_b):
    return pl.pallas_call(
        _final_body,
        grid=(NBLK,),
        in_specs=[
            pl.BlockSpec((1, BLK, D), lambda i: (0, i, 0)),
            pl.BlockSpec((R, BLK, D), lambda i: (0, i, 0)),
            pl.BlockSpec((BLK, 1), lambda i: (i, 0)),
            pl.BlockSpec((R, D, D), lambda i: (0, 0, 0)),
            pl.BlockSpec((1, D, D), lambda i: (i // (NBLK // 2), 0, 0)),
            pl.BlockSpec((1, 1, D), lambda i: (i // (NBLK // 2), 0, 0)),
        ],
        out_specs=pl.BlockSpec((BLK, D), lambda i: (i, 0)),
        out_shape=jax.ShapeDtypeStruct((N, D), jnp.float32),
    )(t1[None], t2, ci[:, None], review_w_stack, fc_w, fc_b)


def kernel(x, review_feat, weight_stack, prob_w_stack, review_w_stack,
           fc_user_w, fc_user_b, fc_item_w, fc_item_b,
           edge_index, edge_rating):
    src = edge_index[0]
    dst = edge_index[1]
    ones_e = jnp.ones((E,), jnp.float32)
    deg = (jax.ops.segment_sum(ones_e, dst, num_segments=N)
           + jax.ops.segment_sum(ones_e, src, num_segments=N))
    ci = lax.rsqrt(jnp.maximum(deg, 1.0))

    wpp = (ci[None, :, None] * (x[None] + weight_stack)).reshape(R * N, D)
    key_src = edge_rating * N + src
    key_dst = edge_rating * N + dst
    c_e = ci[src]

    t1 = jax.ops.segment_sum(wpp[key_src], dst, num_segments=N)
    t2 = jax.ops.segment_sum(c_e[:, None] * review_feat, key_dst,
                             num_segments=R * N).reshape(R, N, D)

    fc_w = jnp.stack([fc_user_w, fc_item_w])
    fc_b = jnp.stack([fc_user_b, fc_item_b])[:, None, :]
    return _final_stage(t1, t2, ci, review_w_stack, fc_w, fc_b)


# speedup vs baseline: 2.0297x; 1.0896x over previous
import functools

import jax
import jax.numpy as jnp
from jax import lax
from jax.experimental import pallas as pl
from jax.experimental.pallas import tpu as pltpu
from jax.experimental.pallas import tpu_sc as plsc

N_U = 5000
N_I = 5000
N = N_U + N_I
E = 320000
D = 128
R = 5

NC = 2
NS = 16
NW = NC * NS
LANES = 16

KA = 80
EPW_A = E // NW
NBLK_A = EPW_A // KA
LA = 632
NPAD_A = LA * NS

KB = 80
EPS_B = E // NS
NBLK_B = EPS_B // KB
SPC = 3
LB = 632
SEGPAD = LB * NS
TRASH = 8
ACC_B = SEGPAD + TRASH

BLK = 1000
NBLK = N // BLK


def _t1_body(wpp_hbm, ks_hbm, dst_hbm, zeros_hbm, out_hbm,
             ks_v, dst_v, rows_v, acc_sh, sem):
    cid = lax.axis_index("c")
    sid = lax.axis_index("s")
    wid = sid * NC + cid
    pltpu.sync_copy(zeros_hbm.at[pl.ds(0, LA)],
                    acc_sh.at[pl.ds(sid * LA, LA)])
    pltpu.sync_copy(ks_hbm.at[wid], ks_v)
    pltpu.sync_copy(dst_hbm.at[wid], dst_v)
    plsc.subcore_barrier()

    def blk(j, carry):
        pltpu.async_copy(wpp_hbm.at[ks_v.at[j]], rows_v, sem).wait()
        pltpu.sync_copy(rows_v, acc_sh.at[dst_v.at[j]], add=True)
        return carry

    lax.fori_loop(0, NBLK_A, blk, 0)
    plsc.subcore_barrier()
    pltpu.sync_copy(acc_sh.at[pl.ds(sid * LA, LA)],
                    out_hbm.at[cid, pl.ds(sid * LA, LA)])


@functools.partial(
    pl.kernel,
    out_type=jax.ShapeDtypeStruct((NC, NPAD_A, D), jnp.float32),
    mesh=plsc.VectorSubcoreMesh(core_axis_name="c", subcore_axis_name="s"),
    scratch_types=[
        pltpu.VMEM((NBLK_A, KA), jnp.int32),
        pltpu.VMEM((NBLK_A, KA), jnp.int32),
        pltpu.VMEM((KA, D), jnp.float32),
        pltpu.VMEM_SHARED((NPAD_A, D), jnp.float32),
        pltpu.SemaphoreType.DMA,
    ],
)
def _t1_sc(*args):
    _t1_body(*args)


def _t2_body(srf_hbm, kd_hbm, zeros_hbm, out_hbm,
             kd_v, buf_v, idx_v, acc_sh, sem):
    cid = lax.axis_index("c")
    sid = lax.axis_index("s")
    pltpu.sync_copy(kd_hbm.at[sid], kd_v)
    for s in range(SPC):
        seg = cid * SPC + s

        @pl.when(seg < R)
        def _():
            base = seg * N
            pltpu.sync_copy(zeros_hbm.at[pl.ds(0, LB)],
                            acc_sh.at[pl.ds(sid * LB, LB)])

            @pl.when(sid == 0)
            def _():
                pltpu.sync_copy(zeros_hbm.at[pl.ds(0, TRASH)],
                                acc_sh.at[pl.ds(SEGPAD, TRASH)])

            plsc.subcore_barrier()

            def blk(j, carry):
                for v in range(KB // LANES):
                    kv = kd_v[j, pl.ds(v * LANES, LANES)]
                    rel = kv - base
                    ok = (rel >= 0) & (rel < N)
                    idx_v[pl.ds(v * LANES, LANES)] = jnp.where(
                        ok, rel, SEGPAD + (kv & (TRASH - 1)))
                e0 = sid * EPS_B + j * KB
                pltpu.sync_copy(srf_hbm.at[pl.ds(e0, KB)], buf_v)
                pltpu.sync_copy(buf_v, acc_sh.at[idx_v], add=True)
                return carry

            lax.fori_loop(0, NBLK_B, blk, 0)
            plsc.subcore_barrier()
            pltpu.sync_copy(acc_sh.at[pl.ds(sid * LB, LB)],
                            out_hbm.at[pl.ds(seg * SEGPAD + sid * LB, LB)])
            plsc.subcore_barrier()


@functools.partial(
    pl.kernel,
    out_type=jax.ShapeDtypeStruct((R * SEGPAD, D), jnp.float32),
    mesh=plsc.VectorSubcoreMesh(core_axis_name="c", subcore_axis_name="s"),
    scratch_types=[
        pltpu.VMEM((NBLK_B, KB), jnp.int32),
        pltpu.VMEM((KB, D), jnp.float32),
        pltpu.VMEM((KB,), jnp.int32),
        pltpu.VMEM_SHARED((ACC_B, D), jnp.float32),
        pltpu.SemaphoreType.DMA,
    ],
)
def _t2_sc(*args):
    _t2_body(*args)


def _wpp_body(x_ref, w_ref, ci_ref, out_ref):
    out_ref[...] = ci_ref[...] * (x_ref[...] + w_ref[0])


def _wpp_stage(x, weight_stack, ci):
    return pl.pallas_call(
        _wpp_body,
        grid=(R, NBLK),
        in_specs=[
            pl.BlockSpec((BLK, D), lambda r, i: (i, 0)),
            pl.BlockSpec((1, BLK, D), lambda r, i: (r, i, 0)),
            pl.BlockSpec((BLK, 1), lambda r, i: (i, 0)),
        ],
        out_specs=pl.BlockSpec((BLK, D), lambda r, i: (r * NBLK + i, 0)),
        out_shape=jax.ShapeDtypeStruct((R * N, D), jnp.float32),
    )(x, weight_stack, ci[:, None])


BLKE = 4000


def _scale_body(rf_ref, ce_ref, out_ref):
    out_ref[...] = ce_ref[...] * rf_ref[...]


def _scale_stage(review_feat, c_e):
    return pl.pallas_call(
        _scale_body,
        grid=(E // BLKE,),
        in_specs=[
            pl.BlockSpec((BLKE, D), lambda i: (i, 0)),
            pl.BlockSpec((BLKE, 1), lambda i: (i, 0)),
        ],
        out_specs=pl.BlockSpec((BLKE, D), lambda i: (i, 0)),
        out_shape=jax.ShapeDtypeStruct((E, D), jnp.float32),
    )(review_feat, c_e[:, None])


def _final_body(t1_ref, t2_ref, ci_ref, rw_ref, fcw_ref, fcb_ref, out_ref):
    acc = t1_ref[0] + t1_ref[1]
    for r in range(R):
        acc = acc + lax.dot_general(
            t2_ref[r], rw_ref[r],
            (((1,), (1,)), ((), ())),
            preferred_element_type=jnp.float32)
    acc = acc * ci_ref[...]
    out_ref[...] = lax.dot_general(
        acc, fcw_ref[0], (((1,), (1,)), ((), ())),
        preferred_element_type=jnp.float32) + fcb_ref[0]


def _final_stage(t1p, t2, ci, review_w_stack, fc_w, fc_b):
    return pl.pallas_call(
        _final_body,
        grid=(NBLK,),
        in_specs=[
            pl.BlockSpec((NC, BLK, D), lambda i: (0, i, 0)),
            pl.BlockSpec((R, BLK, D), lambda i: (0, i, 0)),
            pl.BlockSpec((BLK, 1), lambda i: (i, 0)),
            pl.BlockSpec((R, D, D), lambda i: (0, 0, 0)),
            pl.BlockSpec((1, D, D), lambda i: (i // (NBLK // 2), 0, 0)),
            pl.BlockSpec((1, 1, D), lambda i: (i // (NBLK // 2), 0, 0)),
        ],
        out_specs=pl.BlockSpec((BLK, D), lambda i: (i, 0)),
        out_shape=jax.ShapeDtypeStruct((N, D), jnp.float32),
    )(t1p, t2, ci[:, None], review_w_stack, fc_w, fc_b)


def kernel(x, review_feat, weight_stack, prob_w_stack, review_w_stack,
           fc_user_w, fc_user_b, fc_item_w, fc_item_b,
           edge_index, edge_rating):
    src = edge_index[0]
    dst = edge_index[1]
    ones_e = jnp.ones((E,), jnp.float32)
    deg = (jax.ops.segment_sum(ones_e, dst, num_segments=N)
           + jax.ops.segment_sum(ones_e, src, num_segments=N))
    ci = lax.rsqrt(jnp.maximum(deg, 1.0))

    wpp = _wpp_stage(x, weight_stack, ci)
    srf = _scale_stage(review_feat, ci[src])

    key_src = (edge_rating * N + src).reshape(NW, NBLK_A, KA)
    dst_a = dst.reshape(NW, NBLK_A, KA)
    key_dst = (edge_rating * N + dst).reshape(NS, NBLK_B, KB)
    zeros_nd = jnp.zeros((N, D), jnp.float32)

    t1p = _t1_sc(wpp, key_src, dst_a, zeros_nd)
    t2p = _t2_sc(srf, key_dst, zeros_nd)

    fc_w = jnp.stack([fc_user_w, fc_item_w])
    fc_b = jnp.stack([fc_user_b, fc_item_b])[:, None, :]
    return _final_stage(t1p[:, :N], t2p.reshape(R, SEGPAD, D)[:, :N], ci,
                        review_w_stack, fc_w, fc_b)


# speedup vs baseline: 2.6124x; 1.2871x over previous
import functools

import jax
import jax.numpy as jnp
from jax import lax
from jax.experimental import pallas as pl
from jax.experimental.pallas import tpu as pltpu
from jax.experimental.pallas import tpu_sc as plsc

N_U = 5000
N_I = 5000
N = N_U + N_I
E = 320000
D = 128
R = 5

NC = 2
NS = 16
NW = NC * NS

KA = 80
EPW_A = E // NW
NBLK_A = EPW_A // KA
CHK = 25
NCHK = NBLK_A // CHK
LA = 632
NPAD_A = LA * NS

BLK = 1000
NBLK = N // BLK


def _edge_body(wpp_hbm, srf2_hbm, ks_hbm, dst_hbm, zeros_hbm, out_hbm,
               ks_v, dst_v, rows_v, buf_v, acc_sh, sem):
    cid = lax.axis_index("c")
    sid = lax.axis_index("s")
    wid = sid * NC + cid
    pltpu.sync_copy(zeros_hbm.at[pl.ds(0, LA)],
                    acc_sh.at[pl.ds(sid * LA, LA)])
    plsc.subcore_barrier()

    def chunk(c, carry):
        pltpu.sync_copy(ks_hbm.at[wid, c], ks_v)
        pltpu.sync_copy(dst_hbm.at[wid, c], dst_v)

        def blk(j, carry2):
            cp = pltpu.async_copy(wpp_hbm.at[ks_v.at[j]], rows_v, sem)
            e0 = wid * EPW_A + (c * CHK + j) * KA
            pltpu.sync_copy(srf2_hbm.at[pl.ds(e0, KA)], buf_v)
            pltpu.sync_copy(buf_v, acc_sh.at[dst_v.at[j]], add=True)
            cp.wait()
            pltpu.sync_copy(rows_v, acc_sh.at[dst_v.at[j]], add=True)
            return carry2

        lax.fori_loop(0, CHK, blk, 0)
        return carry

    lax.fori_loop(0, NCHK, chunk, 0)
    plsc.subcore_barrier()
    pltpu.sync_copy(acc_sh.at[pl.ds(sid * LA, LA)],
                    out_hbm.at[cid, pl.ds(sid * LA, LA)])


@functools.partial(
    pl.kernel,
    out_type=jax.ShapeDtypeStruct((NC, NPAD_A, D), jnp.float32),
    mesh=plsc.VectorSubcoreMesh(core_axis_name="c", subcore_axis_name="s"),
    scratch_types=[
        pltpu.VMEM((CHK, KA), jnp.int32),
        pltpu.VMEM((CHK, KA), jnp.int32),
        pltpu.VMEM((KA, D), jnp.float32),
        pltpu.VMEM((KA, D), jnp.float32),
        pltpu.VMEM_SHARED((NPAD_A, D), jnp.float32),
        pltpu.SemaphoreType.DMA,
    ],
)
def _edge_sc(*args):
    _edge_body(*args)


def _wpp_body(x_ref, w_ref, ci_ref, out_ref):
    out_ref[...] = ci_ref[...] * (x_ref[...] + w_ref[0])


def _wpp_stage(x, weight_stack, ci):
    return pl.pallas_call(
        _wpp_body,
        grid=(R, NBLK),
        in_specs=[
            pl.BlockSpec((BLK, D), lambda r, i: (i, 0)),
            pl.BlockSpec((1, BLK, D), lambda r, i: (r, i, 0)),
            pl.BlockSpec((BLK, 1), lambda r, i: (i, 0)),
        ],
        out_specs=pl.BlockSpec((BLK, D), lambda r, i: (r * NBLK + i, 0)),
        out_shape=jax.ShapeDtypeStruct((R * N, D), jnp.float32),
    )(x, weight_stack, ci[:, None])


BLKE = 4000


def _srf2_body(rf_ref, ce_ref, er_ref, w_ref, out_ref):
    srf = ce_ref[...] * rf_ref[...]
    er = er_ref[...]
    acc = jnp.zeros_like(srf)
    for r in range(R):
        m = (er == r).astype(jnp.float32)
        acc = acc + m * lax.dot_general(
            srf, w_ref[r], (((1,), (1,)), ((), ())),
            preferred_element_type=jnp.float32)
    out_ref[...] = acc


def _srf2_stage(review_feat, c_e, edge_rating, review_w_stack):
    return pl.pallas_call(
        _srf2_body,
        grid=(E // BLKE,),
        in_specs=[
            pl.BlockSpec((BLKE, D), lambda i: (i, 0)),
            pl.BlockSpec((BLKE, 1), lambda i: (i, 0)),
            pl.BlockSpec((BLKE, 1), lambda i: (i, 0)),
            pl.BlockSpec((R, D, D), lambda i: (0, 0, 0)),
        ],
        out_specs=pl.BlockSpec((BLKE, D), lambda i: (i, 0)),
        out_shape=jax.ShapeDtypeStruct((E, D), jnp.float32),
    )(review_feat, c_e[:, None], edge_rating[:, None], review_w_stack)


def _final_body(t1_ref, ci_ref, fcw_ref, fcb_ref, out_ref):
    acc = (t1_ref[0] + t1_ref[1]) * ci_ref[...]
    out_ref[...] = lax.dot_general(
        acc, fcw_ref[0], (((1,), (1,)), ((), ())),
        preferred_element_type=jnp.float32) + fcb_ref[0]


def _final_stage(t1p, ci, fc_w, fc_b):
    return pl.pallas_call(
        _final_body,
        grid=(NBLK,),
        in_specs=[
            pl.BlockSpec((NC, BLK, D), lambda i: (0, i, 0)),
            pl.BlockSpec((BLK, 1), lambda i: (i, 0)),
            pl.BlockSpec((1, D, D), lambda i: (i // (NBLK // 2), 0, 0)),
            pl.BlockSpec((1, 1, D), lambda i: (i // (NBLK // 2), 0, 0)),
        ],
        out_specs=pl.BlockSpec((BLK, D), lambda i: (i, 0)),
        out_shape=jax.ShapeDtypeStruct((N, D), jnp.float32),
    )(t1p, ci[:, None], fc_w, fc_b)


def kernel(x, review_feat, weight_stack, prob_w_stack, review_w_stack,
           fc_user_w, fc_user_b, fc_item_w, fc_item_b,
           edge_index, edge_rating):
    src = edge_index[0]
    dst = edge_index[1]
    ones_e = jnp.ones((E,), jnp.float32)
    deg = (jax.ops.segment_sum(ones_e, dst, num_segments=N)
           + jax.ops.segment_sum(ones_e, src, num_segments=N))
    ci = lax.rsqrt(jnp.maximum(deg, 1.0))

    wpp = _wpp_stage(x, weight_stack, ci)
    srf2 = _srf2_stage(review_feat, ci[src], edge_rating,
                       review_w_stack)

    key_src = (edge_rating * N + src).reshape(NW, NCHK, CHK, KA)
    dst_a = dst.reshape(NW, NCHK, CHK, KA)
    zeros_nd = jnp.zeros((N, D), jnp.float32)

    t1p = _edge_sc(wpp, srf2, key_src, dst_a, zeros_nd)

    fc_w = jnp.stack([fc_user_w, fc_item_w])
    fc_b = jnp.stack([fc_user_b, fc_item_b])[:, None, :]
    return _final_stage(t1p[:, :N], ci, fc_w, fc_b)


# speedup vs baseline: 5.1777x; 1.9820x over previous
import functools

import jax
import jax.numpy as jnp
from jax import lax
from jax.experimental import pallas as pl
from jax.experimental.pallas import tpu as pltpu
from jax.experimental.pallas import tpu_sc as plsc

N_U = 5000
N_I = 5000
N = N_U + N_I
E = 320000
D = 128
R = 5

NC = 2
NS = 16
NW = NC * NS

KA = 80
EPW_A = E // NW
NBLK_A = EPW_A // KA
CHK = 25
NCHK = NBLK_A // CHK
LA = 632
NPAD_A = LA * NS

BLK = 1000
NBLK = N // BLK


def _edge_body(wpp_hbm, srf2_hbm, ks_hbm, dst_hbm, zeros_hbm, out_hbm,
               ks_v, dst_v, rows_v, buf_v, acc_sh, sem):
    cid = lax.axis_index("c")
    sid = lax.axis_index("s")
    wid = sid * NC + cid
    pltpu.sync_copy(zeros_hbm.at[pl.ds(0, LA)],
                    acc_sh.at[pl.ds(sid * LA, LA)])
    plsc.subcore_barrier()

    def chunk(c, carry):
        pltpu.sync_copy(ks_hbm.at[wid, c], ks_v)
        pltpu.sync_copy(dst_hbm.at[wid, c], dst_v)

        def blk(j, carry2):
            cp = pltpu.async_copy(wpp_hbm.at[ks_v.at[j]], rows_v, sem)
            e0 = wid * EPW_A + (c * CHK + j) * KA
            pltpu.sync_copy(srf2_hbm.at[pl.ds(e0, KA)], buf_v)
            pltpu.sync_copy(buf_v, acc_sh.at[dst_v.at[j]], add=True)
            cp.wait()
            pltpu.sync_copy(rows_v, acc_sh.at[dst_v.at[j]], add=True)
            return carry2

        lax.fori_loop(0, CHK, blk, 0)
        return carry

    lax.fori_loop(0, NCHK, chunk, 0)
    plsc.subcore_barrier()
    pltpu.sync_copy(acc_sh.at[pl.ds(sid * LA, LA)],
                    out_hbm.at[cid, pl.ds(sid * LA, LA)])


@functools.partial(
    pl.kernel,
    out_type=jax.ShapeDtypeStruct((NC, NPAD_A, D), jnp.float32),
    mesh=plsc.VectorSubcoreMesh(core_axis_name="c", subcore_axis_name="s"),
    scratch_types=[
        pltpu.VMEM((CHK, KA), jnp.int32),
        pltpu.VMEM((CHK, KA), jnp.int32),
        pltpu.VMEM((KA, D), jnp.float32),
        pltpu.VMEM((KA, D), jnp.float32),
        pltpu.VMEM_SHARED((NPAD_A, D), jnp.float32),
        pltpu.SemaphoreType.DMA,
    ],
)
def _edge_sc(*args):
    _edge_body(*args)


VR = 16
NVG = EPW_A // VR


def _cig_body(ci_hbm, src_hbm, out_hbm, ci_v, src_v, out_v):
    cid = lax.axis_index("c")
    sid = lax.axis_index("s")
    wid = sid * NC + cid
    pltpu.sync_copy(ci_hbm, ci_v)
    pltpu.sync_copy(src_hbm.at[wid], src_v)

    def blk(j, carry):
        idx = src_v[0, pl.ds(j * VR, VR)]
        row = lax.shift_right_logical(idx, 7)
        col = lax.bitwise_and(idx, 127)
        out_v[0, pl.ds(j * VR, VR)] = plsc.load_gather(ci_v, [row, col])
        return carry

    lax.fori_loop(0, NVG, blk, 0)
    pltpu.sync_copy(out_v, out_hbm.at[wid])


@functools.partial(
    pl.kernel,
    out_type=jax.ShapeDtypeStruct((NW, 1, EPW_A), jnp.float32),
    mesh=plsc.VectorSubcoreMesh(core_axis_name="c", subcore_axis_name="s"),
    compiler_params=pltpu.CompilerParams(needs_layout_passes=False),
    scratch_types=[
        pltpu.VMEM((N // 128 + 1, 128), jnp.float32),
        pltpu.VMEM((1, EPW_A), jnp.int32),
        pltpu.VMEM((1, EPW_A), jnp.float32),
    ],
)
def _cig_sc(*args):
    _cig_body(*args)


def _wpp_body(x_ref, w_ref, ci_ref, out_ref):
    out_ref[...] = ci_ref[...] * (x_ref[...] + w_ref[0])


def _wpp_stage(x, weight_stack, ci):
    return pl.pallas_call(
        _wpp_body,
        grid=(R, NBLK),
        in_specs=[
            pl.BlockSpec((BLK, D), lambda r, i: (i, 0)),
            pl.BlockSpec((1, BLK, D), lambda r, i: (r, i, 0)),
            pl.BlockSpec((BLK, 1), lambda r, i: (i, 0)),
        ],
        out_specs=pl.BlockSpec((BLK, D), lambda r, i: (r * NBLK + i, 0)),
        out_shape=jax.ShapeDtypeStruct((R * N, D), jnp.float32),
    )(x, weight_stack, ci[:, None])


BLKE = 4000


def _srf2_body(rf_ref, ce_ref, er_ref, w_ref, out_ref):
    srf = ce_ref[...] * rf_ref[...]
    er = er_ref[...]
    acc = jnp.zeros_like(srf)
    for r in range(R):
        m = (er == r).astype(jnp.float32)
        acc = acc + m * lax.dot_general(
            srf, w_ref[r], (((1,), (1,)), ((), ())),
            preferred_element_type=jnp.float32)
    out_ref[...] = acc


def _srf2_stage(review_feat, c_e, edge_rating, review_w_stack):
    return pl.pallas_call(
        _srf2_body,
        grid=(E // BLKE,),
        in_specs=[
            pl.BlockSpec((BLKE, D), lambda i: (i, 0)),
            pl.BlockSpec((BLKE, 1), lambda i: (i, 0)),
            pl.BlockSpec((BLKE, 1), lambda i: (i, 0)),
            pl.BlockSpec((R, D, D), lambda i: (0, 0, 0)),
        ],
        out_specs=pl.BlockSpec((BLKE, D), lambda i: (i, 0)),
        out_shape=jax.ShapeDtypeStruct((E, D), jnp.float32),
    )(review_feat, c_e[:, None], edge_rating[:, None], review_w_stack)


def _final_body(t1_ref, ci_ref, fcw_ref, fcb_ref, out_ref):
    acc = (t1_ref[0] + t1_ref[1]) * ci_ref[...]
    out_ref[...] = lax.dot_general(
        acc, fcw_ref[0], (((1,), (1,)), ((), ())),
        preferred_element_type=jnp.float32) + fcb_ref[0]


def _final_stage(t1p, ci, fc_w, fc_b):
    return pl.pallas_call(
        _final_body,
        grid=(NBLK,),
        in_specs=[
            pl.BlockSpec((NC, BLK, D), lambda i: (0, i, 0)),
            pl.BlockSpec((BLK, 1), lambda i: (i, 0)),
            pl.BlockSpec((1, D, D), lambda i: (i // (NBLK // 2), 0, 0)),
            pl.BlockSpec((1, 1, D), lambda i: (i // (NBLK // 2), 0, 0)),
        ],
        out_specs=pl.BlockSpec((BLK, D), lambda i: (i, 0)),
        out_shape=jax.ShapeDtypeStruct((N, D), jnp.float32),
    )(t1p, ci[:, None], fc_w, fc_b)


def kernel(x, review_feat, weight_stack, prob_w_stack, review_w_stack,
           fc_user_w, fc_user_b, fc_item_w, fc_item_b,
           edge_index, edge_rating):
    src = edge_index[0]
    dst = edge_index[1]
    ones_e = jnp.ones((E,), jnp.float32)
    deg = (jax.ops.segment_sum(ones_e, dst, num_segments=N)
           + jax.ops.segment_sum(ones_e, src, num_segments=N))
    ci = lax.rsqrt(jnp.maximum(deg, 1.0))

    ci_pad = jnp.pad(ci, (0, 128 - N % 128)).reshape(N // 128 + 1, 128)
    ci_e = _cig_sc(ci_pad, src.reshape(NW, 1, EPW_A)).reshape(E)
    wpp = _wpp_stage(x, weight_stack, ci)
    srf2 = _srf2_stage(review_feat, ci_e, edge_rating,
                       review_w_stack)

    key_src = (edge_rating * N + src).reshape(NW, NCHK, CHK, KA)
    dst_a = dst.reshape(NW, NCHK, CHK, KA)
    zeros_nd = jnp.zeros((N, D), jnp.float32)

    t1p = _edge_sc(wpp, srf2, key_src, dst_a, zeros_nd)

    fc_w = jnp.stack([fc_user_w, fc_item_w])
    fc_b = jnp.stack([fc_user_b, fc_item_b])[:, None, :]
    return _final_stage(t1p[:, :N], ci, fc_w, fc_b)
